# baseline (speedup 1.0000x reference)
import functools

import jax
import jax.numpy as jnp
from jax import lax
from jax.experimental import pallas as pl
from jax.experimental.pallas import tpu as pltpu
from jax.experimental.pallas import tpu_sc as plsc

_EPS = 1e-5
_NC = 2
_NS = 16
_NW = _NC * _NS
_CHUNK = 128


def _sc_gather(table, idx2d, latent):
    Bb, L = idx2d.shape
    rows_per_w = Bb // _NW
    c1 = min(L, _CHUNK)
    c2 = L - c1
    n_idx = Bb * L
    mesh = plsc.VectorSubcoreMesh(core_axis_name="c", subcore_axis_name="s")

    @functools.partial(
        pl.kernel,
        mesh=mesh,
        out_type=jax.ShapeDtypeStruct((n_idx, latent), jnp.float32),
        scratch_types=[
            pltpu.VMEM((rows_per_w, L), jnp.int32),
            pltpu.VMEM((2, c1, latent), jnp.float32),
            pltpu.VMEM((2, c2, latent), jnp.float32),
            pltpu.SemaphoreType.DMA((2,)),
            pltpu.SemaphoreType.DMA((2,)),
        ],
        compiler_params=pltpu.CompilerParams(use_tc_tiling_on_sc=False),
    )
    def k(table_hbm, idx_hbm, out_hbm, idx_v, bufA, bufB, semA, semB):
        wid = lax.axis_index("s") * _NC + lax.axis_index("c")
        row0 = wid * rows_per_w
        base = row0 * L
        pltpu.sync_copy(idx_hbm.at[pl.ds(row0, rows_per_w)], idx_v)

        def start(r, slot):
            pltpu.async_copy(table_hbm.at[idx_v.at[r, pl.ds(0, c1)]],
                             bufA.at[slot], semA.at[slot])
            pltpu.async_copy(table_hbm.at[idx_v.at[r, pl.ds(c1, c2)]],
                             bufB.at[slot], semB.at[slot])

        def wait(r, slot):
            pltpu.make_async_copy(table_hbm.at[idx_v.at[r, pl.ds(0, c1)]],
                                  bufA.at[slot], semA.at[slot]).wait()
            pltpu.make_async_copy(table_hbm.at[idx_v.at[r, pl.ds(c1, c2)]],
                                  bufB.at[slot], semB.at[slot]).wait()

        start(0, 0)

        def body(r, carry):
            slot = lax.rem(r, 2)

            @pl.when(r + 1 < rows_per_w)
            def _():
                start(r + 1, lax.rem(r + 1, 2))

            wait(r, slot)
            pltpu.sync_copy(bufA.at[slot], out_hbm.at[pl.ds(base + r * L, c1)])
            pltpu.sync_copy(bufB.at[slot],
                            out_hbm.at[pl.ds(base + r * L + c1, c2)])
            return carry

        lax.fori_loop(0, rows_per_w, body, 0)

    return k(table, idx2d)


def _tc_dense(seq_rep, aa2w, Wst, bs, Wtt, bt):
    B, L, D = seq_rep.shape
    latent = aa2w.shape[-1] // 2
    H = L // 2
    bB = 32
    grid = (B // bB,)

    inv_d = 1.0 / D

    def body(seq_ref, aa_ref, wst_ref, bs_ref, wtt_ref, bt_ref,
             ones_ref, out_ref):
        x = seq_ref[...]
        xf = x.reshape(bB * L, D)
        ones_w = ones_ref[...]
        s1 = jnp.dot(xf, ones_w, preferred_element_type=jnp.float32)
        s2 = jnp.dot(xf * xf, ones_w, preferred_element_type=jnp.float32)
        m = s1[:, :1] * inv_d
        var = s2[:, :1] * inv_d - m * m
        r = lax.rsqrt(var + _EPS)
        xn = (xf - m) * r
        tok = jnp.dot(xn, wtt_ref[...], preferred_element_type=jnp.float32)
        tok = (tok + bt_ref[...]).reshape(bB, H, 2, latent)
        sm = jnp.mean(x, axis=1)
        m2 = jnp.mean(sm, axis=-1, keepdims=True)
        sc = sm - m2
        v2 = jnp.mean(sc * sc, axis=-1, keepdims=True)
        sn = sc * lax.rsqrt(v2 + _EPS)
        se = jnp.dot(sn, wst_ref[...], preferred_element_type=jnp.float32)
        se = (se + bs_ref[...]).reshape(bB, 1, latent)
        aa = aa_ref[...].reshape(bB, H, 2 * latent)
        out_ref[:, :, :latent] = tok[:, :, 0, :] + se + aa[:, :, :latent]
        out_ref[:, :, latent:] = tok[:, :, 1, :] + se + aa[:, :, latent:]

    return pl.pallas_call(
        body,
        grid=grid,
        in_specs=[
            pl.BlockSpec((bB, L, D), lambda i: (i, 0, 0)),
            pl.BlockSpec((bB * H, 2 * latent), lambda i: (i, 0)),
            pl.BlockSpec((D, latent), lambda i: (0, 0)),
            pl.BlockSpec((1, latent), lambda i: (0, 0)),
            pl.BlockSpec((D, latent), lambda i: (0, 0)),
            pl.BlockSpec((1, latent), lambda i: (0, 0)),
            pl.BlockSpec((D, 8), lambda i: (0, 0)),
        ],
        out_specs=pl.BlockSpec((bB, H, 2 * latent), lambda i: (i, 0, 0)),
        out_shape=jax.ShapeDtypeStruct((B, H, 2 * latent), jnp.float32),
    )(seq_rep, aa2w, Wst, bs, Wtt, bt, jnp.ones((D, 8), jnp.float32))


def kernel(aa_types, seq_rep, aa_table, W_seq, b_seq, W_tok, b_tok,
           g_seq, be_seq, g_tok, be_tok):
    B, L, D = seq_rep.shape
    latent = aa_table.shape[-1]
    aa_flat = _sc_gather(aa_table, aa_types.astype(jnp.int32), latent)
    aa2w = aa_flat.reshape(B * L // 2, 2 * latent)
    Wg_seq = g_seq[:, None] * W_seq.T
    b2_seq = be_seq @ W_seq.T + b_seq
    Wg_tok = g_tok[:, None] * W_tok.T
    b2_tok = be_tok @ W_tok.T + b_tok
    out2w = _tc_dense(
        seq_rep, aa2w,
        Wg_seq, b2_seq.reshape(1, -1),
        Wg_tok, b2_tok.reshape(1, -1),
    )
    return out2w.reshape(B, L, latent)


# speedup vs baseline: 1.2975x; 1.2975x over previous
import functools

import jax
import jax.numpy as jnp
from jax import lax
from jax.experimental import pallas as pl
from jax.experimental.pallas import tpu as pltpu
from jax.experimental.pallas import tpu_sc as plsc

_EPS = 1e-5
_NC = 2
_NS = 16
_NW = _NC * _NS
_CHUNK = 128


def _sc_gather(table, idx2d, latent):
    Bb, L = idx2d.shape
    rows_per_w = Bb // _NW
    c1 = min(L, _CHUNK)
    c2 = L - c1
    n_idx = Bb * L
    mesh = plsc.VectorSubcoreMesh(core_axis_name="c", subcore_axis_name="s")

    @functools.partial(
        pl.kernel,
        mesh=mesh,
        out_type=jax.ShapeDtypeStruct((n_idx, latent), jnp.float32),
        scratch_types=[
            pltpu.VMEM((rows_per_w, L), jnp.int32),
            pltpu.VMEM((2, c1, latent), jnp.float32),
            pltpu.VMEM((2, c2, latent), jnp.float32),
            pltpu.SemaphoreType.DMA((2,)),
            pltpu.SemaphoreType.DMA((2,)),
        ],
        compiler_params=pltpu.CompilerParams(use_tc_tiling_on_sc=False),
    )
    def k(table_hbm, idx_hbm, out_hbm, idx_v, bufA, bufB, semA, semB):
        wid = lax.axis_index("s") * _NC + lax.axis_index("c")
        row0 = wid * rows_per_w
        base = row0 * L
        pltpu.sync_copy(idx_hbm.at[pl.ds(row0, rows_per_w)], idx_v)

        def start(r, slot):
            pltpu.async_copy(table_hbm.at[idx_v.at[r, pl.ds(0, c1)]],
                             bufA.at[slot], semA.at[slot])
            pltpu.async_copy(table_hbm.at[idx_v.at[r, pl.ds(c1, c2)]],
                             bufB.at[slot], semB.at[slot])

        def wait(r, slot):
            pltpu.make_async_copy(table_hbm.at[idx_v.at[r, pl.ds(0, c1)]],
                                  bufA.at[slot], semA.at[slot]).wait()
            pltpu.make_async_copy(table_hbm.at[idx_v.at[r, pl.ds(c1, c2)]],
                                  bufB.at[slot], semB.at[slot]).wait()

        start(0, 0)

        def body(r, carry):
            slot = lax.rem(r, 2)

            @pl.when(r + 1 < rows_per_w)
            def _():
                start(r + 1, lax.rem(r + 1, 2))

            wait(r, slot)
            pltpu.sync_copy(bufA.at[slot], out_hbm.at[pl.ds(base + r * L, c1)])
            pltpu.sync_copy(bufB.at[slot],
                            out_hbm.at[pl.ds(base + r * L + c1, c2)])
            return carry

        lax.fori_loop(0, rows_per_w, body, 0)

    return k(table, idx2d)


def _tc_dense(seq_rep, aa2w, Wst, bs, Wtt, bt):
    B, L, D = seq_rep.shape
    latent = aa2w.shape[-1] // 2
    H = L // 2
    bB = 32
    grid = (B // bB,)

    def body(seq_ref, aa_ref, wst_ref, bs_ref, wtt_ref, bt_ref, out_ref):
        x = seq_ref[...]
        m = jnp.mean(x, axis=-1, keepdims=True)
        xc = x - m
        v = jnp.mean(xc * xc, axis=-1, keepdims=True)
        xn = xc * lax.rsqrt(v + _EPS)
        tok = jnp.dot(xn.reshape(bB * L, D), wtt_ref[...],
                      preferred_element_type=jnp.float32)
        sm = jnp.mean(x, axis=1)
        m2 = jnp.mean(sm, axis=-1, keepdims=True)
        sc = sm - m2
        v2 = jnp.mean(sc * sc, axis=-1, keepdims=True)
        sn = sc * lax.rsqrt(v2 + _EPS)
        se = jnp.dot(sn, wst_ref[...], preferred_element_type=jnp.float32)
        se = (se + bs_ref[...] + bt_ref[...]).reshape(bB, 1, latent)
        tot = tok.reshape(bB, L, latent) + se
        totp = tot.reshape(bB, H, 2, latent)
        pk = jnp.concatenate([totp[:, :, 0, :], totp[:, :, 1, :]], axis=-1)
        out_ref[...] = pk + aa_ref[...].reshape(bB, H, 2 * latent)

    return pl.pallas_call(
        body,
        grid=grid,
        in_specs=[
            pl.BlockSpec((bB, L, D), lambda i: (i, 0, 0)),
            pl.BlockSpec((bB * H, 2 * latent), lambda i: (i, 0)),
            pl.BlockSpec((D, latent), lambda i: (0, 0)),
            pl.BlockSpec((1, latent), lambda i: (0, 0)),
            pl.BlockSpec((D, latent), lambda i: (0, 0)),
            pl.BlockSpec((1, latent), lambda i: (0, 0)),
        ],
        out_specs=pl.BlockSpec((bB, H, 2 * latent), lambda i: (i, 0, 0)),
        out_shape=jax.ShapeDtypeStruct((B, H, 2 * latent), jnp.float32),
    )(seq_rep, aa2w, Wst, bs, Wtt, bt)


def kernel(aa_types, seq_rep, aa_table, W_seq, b_seq, W_tok, b_tok,
           g_seq, be_seq, g_tok, be_tok):
    B, L, D = seq_rep.shape
    latent = aa_table.shape[-1]
    aa_flat = _sc_gather(aa_table, aa_types.astype(jnp.int32), latent)
    aa2w = aa_flat.reshape(B * L // 2, 2 * latent)
    Wg_seq = g_seq[:, None] * W_seq.T
    b2_seq = be_seq @ W_seq.T + b_seq
    Wg_tok = g_tok[:, None] * W_tok.T
    b2_tok = be_tok @ W_tok.T + b_tok
    out2w = _tc_dense(
        seq_rep, aa2w,
        Wg_seq, b2_seq.reshape(1, -1),
        Wg_tok, b2_tok.reshape(1, -1),
    )
    return out2w.reshape(B, L, latent)
